# speedup vs baseline: 1.5162x; 1.5162x over previous
import functools

import jax
import jax.numpy as jnp
from jax import lax
from jax.experimental import pallas as pl
from jax.experimental.pallas import tpu as pltpu
from jax.experimental.pallas import tpu_sc as plsc


def _argmax_body(att_ref, idx_ref, *, t_in):
    b_dim = att_ref.shape[0]
    for b in range(b_dim):
        a = att_ref[b]
        m = jnp.max(a, axis=1, keepdims=True)
        ii = lax.broadcasted_iota(jnp.int32, a.shape, 1)
        cand = jnp.where(a == m, ii, jnp.int32(t_in))
        idx_ref[b] = jnp.min(cand, axis=1) + jnp.int32(b * t_in)


def _argmax_global(att, t_chunk):
    b_dim, t_out, t_in = att.shape
    grid = (t_out // t_chunk,)
    return pl.pallas_call(
        functools.partial(_argmax_body, t_in=t_in),
        grid=grid,
        in_specs=[
            pl.BlockSpec((b_dim, t_chunk, t_in), lambda i: (0, i, 0)),
        ],
        out_specs=pl.BlockSpec((b_dim, t_chunk), lambda i: (0, i)),
        out_shape=jax.ShapeDtypeStruct((b_dim, t_out), jnp.int32),
    )(att)


def _make_sc_gather(n_rows, d, nc, ns):
    nw = nc * ns
    rows_per_w = n_rows // nw
    chunk = 128
    k_steps = rows_per_w // chunk
    mesh = plsc.VectorSubcoreMesh(core_axis_name="c", subcore_axis_name="s")

    @functools.partial(
        pl.kernel,
        mesh=mesh,
        out_type=jax.ShapeDtypeStruct((n_rows, d), jnp.float32),
        scratch_types=[
            pltpu.VMEM((k_steps, chunk), jnp.int32),
            pltpu.VMEM((rows_per_w, d), jnp.float32),
            pltpu.SemaphoreType.DMA,
        ],
    )
    def gather(y_hbm, idx_hbm, out_hbm, idx_v, rows_v, sem):
        wid = lax.axis_index("s") * nc + lax.axis_index("c")
        base = wid * rows_per_w
        pltpu.sync_copy(idx_hbm.at[pl.ds(wid * k_steps, k_steps)], idx_v)
        copies = []
        for k in range(k_steps):
            copies.append(
                pltpu.async_copy(
                    y_hbm.at[idx_v.at[k]],
                    rows_v.at[pl.ds(k * chunk, chunk)],
                    sem,
                )
            )
        for c in copies:
            c.wait()
        pltpu.sync_copy(rows_v, out_hbm.at[pl.ds(base, rows_per_w)])

    return gather


def kernel(y, att):
    b_dim, t_out, t_in = att.shape
    d = y.shape[2]
    n_rows = b_dim * t_out

    idx = _argmax_global(att, t_chunk=256)

    info = plsc.get_sparse_core_info()
    nc, ns = info.num_cores, info.num_subcores
    nw = nc * ns
    k_steps = (n_rows // nw) // 128

    y_flat = y.reshape(n_rows, d)
    idx_2d = idx.reshape(nw * k_steps, 128)
    out_flat = _make_sc_gather(n_rows, d, nc, ns)(y_flat, idx_2d)
    return out_flat.reshape(b_dim, t_out, d)


# speedup vs baseline: 1.5482x; 1.0211x over previous
import functools

import jax
import jax.numpy as jnp
from jax import lax
from jax.experimental import pallas as pl
from jax.experimental.pallas import tpu as pltpu
from jax.experimental.pallas import tpu_sc as plsc


def _argmax_body(att_ref, idx_ref, *, t_in):
    b_dim = att_ref.shape[0]
    tf = jnp.float32(t_in)
    shape2 = att_ref.shape[1:]
    fi = tf - lax.broadcasted_iota(jnp.int32, shape2, 1).astype(jnp.float32)
    for b in range(b_dim):
        a = att_ref[b]
        m = jnp.max(a, axis=1, keepdims=True)
        cand = jnp.where(a == m, fi, jnp.float32(0.0))
        r = jnp.max(cand, axis=1)
        idx_ref[b] = (tf - r).astype(jnp.int32) + jnp.int32(b * t_in)


def _argmax_global(att, t_chunk):
    b_dim, t_out, t_in = att.shape
    grid = (t_out // t_chunk,)
    return pl.pallas_call(
        functools.partial(_argmax_body, t_in=t_in),
        grid=grid,
        in_specs=[
            pl.BlockSpec((b_dim, t_chunk, t_in), lambda i: (0, i, 0)),
        ],
        out_specs=pl.BlockSpec((b_dim, t_chunk), lambda i: (0, i)),
        out_shape=jax.ShapeDtypeStruct((b_dim, t_out), jnp.int32),
    )(att)


def _make_sc_gather(n_rows, d, nc, ns):
    nw = nc * ns
    rows_per_w = n_rows // nw
    chunk = 128
    k_steps = rows_per_w // chunk
    mesh = plsc.VectorSubcoreMesh(core_axis_name="c", subcore_axis_name="s")

    @functools.partial(
        pl.kernel,
        mesh=mesh,
        out_type=jax.ShapeDtypeStruct((n_rows, d), jnp.float32),
        scratch_types=[
            pltpu.VMEM((k_steps, chunk), jnp.int32),
            pltpu.VMEM((rows_per_w, d), jnp.float32),
            pltpu.SemaphoreType.DMA,
        ],
    )
    def gather(y_hbm, idx_hbm, out_hbm, idx_v, rows_v, sem):
        wid = lax.axis_index("s") * nc + lax.axis_index("c")
        base = wid * rows_per_w
        pltpu.sync_copy(idx_hbm.at[pl.ds(wid * k_steps, k_steps)], idx_v)
        copies = []
        for k in range(k_steps):
            copies.append(
                pltpu.async_copy(
                    y_hbm.at[idx_v.at[k]],
                    rows_v.at[pl.ds(k * chunk, chunk)],
                    sem,
                )
            )
        for c in copies:
            c.wait()
        pltpu.sync_copy(rows_v, out_hbm.at[pl.ds(base, rows_per_w)])

    return gather


def kernel(y, att):
    b_dim, t_out, t_in = att.shape
    d = y.shape[2]
    n_rows = b_dim * t_out

    idx = _argmax_global(att, t_chunk=256)

    info = plsc.get_sparse_core_info()
    nc, ns = info.num_cores, info.num_subcores
    nw = nc * ns
    k_steps = (n_rows // nw) // 128

    y_flat = y.reshape(n_rows, d)
    idx_2d = idx.reshape(nw * k_steps, 128)
    out_flat = _make_sc_gather(n_rows, d, nc, ns)(y_flat, idx_2d)
    return out_flat.reshape(b_dim, t_out, d)


# speedup vs baseline: 1.5874x; 1.0253x over previous
import functools

import jax
import jax.numpy as jnp
from jax import lax
from jax.experimental import pallas as pl
from jax.experimental.pallas import tpu as pltpu
from jax.experimental.pallas import tpu_sc as plsc


def _argmax_body(att_ref, idx_ref, *, t_in):
    b_dim = att_ref.shape[0]
    tf = jnp.float32(t_in)
    shape2 = att_ref.shape[1:]
    fi = tf - lax.broadcasted_iota(jnp.int32, shape2, 1).astype(jnp.float32)
    for b in range(b_dim):
        a = att_ref[b]
        m = jnp.max(a, axis=1, keepdims=True)
        cand = jnp.where(a == m, fi, jnp.float32(0.0))
        r = jnp.max(cand, axis=1)
        idx_ref[b] = (tf - r).astype(jnp.int32) + jnp.int32(b * t_in)


def _argmax_global(att, t_chunk):
    b_dim, t_out, t_in = att.shape
    grid = (t_out // t_chunk,)
    return pl.pallas_call(
        functools.partial(_argmax_body, t_in=t_in),
        grid=grid,
        in_specs=[
            pl.BlockSpec((b_dim, t_chunk, t_in), lambda i: (0, i, 0)),
        ],
        out_specs=pl.BlockSpec((b_dim, t_chunk), lambda i: (0, i)),
        out_shape=jax.ShapeDtypeStruct((b_dim, t_out), jnp.int32),
    )(att)


def _make_sc_gather(n_rows, d, nc, ns):
    nw = nc * ns
    rows_per_w = n_rows // nw
    chunk = 128
    k_steps = rows_per_w // chunk
    mesh = plsc.VectorSubcoreMesh(core_axis_name="c", subcore_axis_name="s")

    @functools.partial(
        pl.kernel,
        mesh=mesh,
        out_type=jax.ShapeDtypeStruct((n_rows, d), jnp.float32),
        scratch_types=[
            pltpu.VMEM((k_steps, chunk), jnp.int32),
            pltpu.VMEM((rows_per_w, d), jnp.float32),
            pltpu.SemaphoreType.DMA,
        ],
    )
    def gather(y_hbm, idx_hbm, out_hbm, idx_v, rows_v, sem):
        wid = lax.axis_index("s") * nc + lax.axis_index("c")
        base = wid * rows_per_w
        pltpu.sync_copy(idx_hbm.at[pl.ds(wid * k_steps, k_steps)], idx_v)
        copies = []
        for k in range(k_steps):
            copies.append(
                pltpu.async_copy(
                    y_hbm.at[idx_v.at[k]],
                    rows_v.at[pl.ds(k * chunk, chunk)],
                    sem,
                )
            )
        for c in copies:
            c.wait()
        pltpu.sync_copy(rows_v, out_hbm.at[pl.ds(base, rows_per_w)])

    return gather


def kernel(y, att):
    b_dim, t_out, t_in = att.shape
    d = y.shape[2]
    n_rows = b_dim * t_out

    idx = _argmax_global(att, t_chunk=128)

    info = plsc.get_sparse_core_info()
    nc, ns = info.num_cores, info.num_subcores
    nw = nc * ns
    k_steps = (n_rows // nw) // 128

    y_flat = y.reshape(n_rows, d)
    idx_2d = idx.reshape(nw * k_steps, 128)
    out_flat = _make_sc_gather(n_rows, d, nc, ns)(y_flat, idx_2d)
    return out_flat.reshape(b_dim, t_out, d)
